# baseline (speedup 1.0000x reference)
import jax
import jax.numpy as jnp
from jax.experimental import pallas as pl
from jax.experimental.pallas import tpu as pltpu

_B = 128
_V = 100000
_BR = 16
_NCH = _B // _BR


def _chunk_copy(x_hbm, buf, sem, j):
    return pltpu.make_async_copy(
        x_hbm.at[pl.ds(j * _BR, _BR), :],
        buf,
        sem.at[j],
    )


def _lse_pick_kernel(a_ref, x_hbm, o_ref, *bufs_and_sem):
    bufs = bufs_and_sem[:_NCH]
    sem = bufs_and_sem[_NCH]
    for j in range(_NCH):
        _chunk_copy(x_hbm, bufs[j], sem, j).start()

    col = jax.lax.broadcasted_iota(jnp.int32, (_BR, _V), 1)
    for j in range(_NCH):
        _chunk_copy(x_hbm, bufs[j], sem, j).wait()
        x = bufs[j][...]
        a = a_ref[pl.ds(j * _BR, _BR), :]
        m = jnp.max(x, axis=-1, keepdims=True)
        s = jnp.sum(jnp.exp(x - m), axis=-1, keepdims=True)
        pick = jnp.sum(jnp.where(col == a, x, 0.0), axis=-1, keepdims=True)
        o_ref[pl.ds(j * _BR, _BR), :] = pick - (m + jnp.log(s))


@jax.jit
def kernel(logits, actions):
    out = pl.pallas_call(
        _lse_pick_kernel,
        in_specs=[
            pl.BlockSpec(memory_space=pltpu.MemorySpace.VMEM),
            pl.BlockSpec(memory_space=pltpu.MemorySpace.HBM),
        ],
        out_specs=pl.BlockSpec(memory_space=pltpu.MemorySpace.VMEM),
        out_shape=jax.ShapeDtypeStruct((_B, 1), jnp.float32),
        scratch_shapes=[pltpu.VMEM((_BR, _V), jnp.float32) for _ in range(_NCH)]
        + [pltpu.SemaphoreType.DMA((_NCH,))],
    )(actions, logits)
    return out
